# Initial kernel scaffold; baseline (speedup 1.0000x reference)
import jax
import jax.numpy as jnp
from jax.experimental import pallas as pl


def kernel(nodes, labels):
    raise NotImplementedError("write your pallas kernel here")


# speedup vs baseline: 3.6226x; 3.6226x over previous
import functools

import jax
import jax.numpy as jnp
from jax.experimental import pallas as pl

N = 2048
D = 256
K1 = 11
BLK = 256
NB = N // BLK
_BIG = 3.4e38


def _w_kernel(nodes_ref, w_ref):
    i = pl.program_id(0)
    nodes = nodes_ref[...]
    blk = nodes_ref[pl.ds(i * BLK, BLK), :]
    sq = jnp.sum(nodes * nodes, axis=1)
    sq_blk = jnp.sum(blk * blk, axis=1)
    g = jax.lax.dot_general(
        blk, nodes, (((1,), (1,)), ((), ())),
        preferred_element_type=jnp.float32,
        precision=jax.lax.Precision.HIGHEST,
    )
    s = sq_blk[:, None] + sq[None, :] - 2.0 * g
    s = jnp.maximum(s, 0.0)
    gi = jax.lax.broadcasted_iota(jnp.int32, (BLK, N), 0) + i * BLK
    gj = jax.lax.broadcasted_iota(jnp.int32, (BLK, N), 1)
    s = jnp.where(gi == gj, 0.0, s)
    cur = s
    for _ in range(K1 - 1):
        m = jnp.min(cur, axis=1, keepdims=True)
        cur = jnp.where(cur == m, _BIG, cur)
    thr = jnp.min(cur, axis=1, keepdims=True)
    w_ref[...] = jnp.where(s <= thr, jnp.exp(s * -0.5), 0.0)


def _a_deg_kernel(wij_ref, wji_ref, a_ref, deg_ref):
    j = pl.program_id(1)
    a = jnp.maximum(wij_ref[...], wji_ref[...].T)
    a_ref[...] = a
    part = jnp.sum(a, axis=1)[None, None, :]

    @pl.when(j == 0)
    def _():
        deg_ref[...] = part

    @pl.when(j != 0)
    def _():
        deg_ref[...] += part


def _lsym_kernel(a_ref, degi_ref, degj_ref, out_ref):
    i = pl.program_id(0)
    j = pl.program_id(1)
    a = a_ref[...]
    degi = degi_ref[0, 0, :]
    degj = degj_ref[0, 0, :]
    dinv_i = 1.0 / jnp.sqrt(degi)
    dinv_j = 1.0 / jnp.sqrt(degj)
    gi = jax.lax.broadcasted_iota(jnp.int32, (BLK, BLK), 0) + i * BLK
    gj = jax.lax.broadcasted_iota(jnp.int32, (BLK, BLK), 1) + j * BLK
    l = jnp.where(gi == gj, degi[:, None] - a, -a)
    m1 = (dinv_i[:, None] * l) * dinv_j[None, :]
    m2 = (dinv_j[None, :] * l) * dinv_i[:, None]
    out_ref[...] = 0.5 * (m1 + m2)


@functools.partial(jax.jit, static_argnames=("interpret",))
def _build_lsym(nodes, interpret=False):
    w = pl.pallas_call(
        _w_kernel,
        grid=(NB,),
        in_specs=[pl.BlockSpec((N, D), lambda i: (0, 0))],
        out_specs=pl.BlockSpec((BLK, N), lambda i: (i, 0)),
        out_shape=jax.ShapeDtypeStruct((N, N), jnp.float32),
        interpret=interpret,
    )(nodes)
    a, deg3 = pl.pallas_call(
        _a_deg_kernel,
        grid=(NB, NB),
        in_specs=[
            pl.BlockSpec((BLK, BLK), lambda i, j: (i, j)),
            pl.BlockSpec((BLK, BLK), lambda i, j: (j, i)),
        ],
        out_specs=[
            pl.BlockSpec((BLK, BLK), lambda i, j: (i, j)),
            pl.BlockSpec((1, 1, BLK), lambda i, j: (i, 0, 0)),
        ],
        out_shape=[
            jax.ShapeDtypeStruct((N, N), jnp.float32),
            jax.ShapeDtypeStruct((NB, 1, BLK), jnp.float32),
        ],
        interpret=interpret,
    )(w, w)
    lsym = pl.pallas_call(
        _lsym_kernel,
        grid=(NB, NB),
        in_specs=[
            pl.BlockSpec((BLK, BLK), lambda i, j: (i, j)),
            pl.BlockSpec((1, 1, BLK), lambda i, j: (i, 0, 0)),
            pl.BlockSpec((1, 1, BLK), lambda i, j: (j, 0, 0)),
        ],
        out_specs=pl.BlockSpec((BLK, BLK), lambda i, j: (i, j)),
        out_shape=jax.ShapeDtypeStruct((N, N), jnp.float32),
        interpret=interpret,
    )(a, deg3, deg3)
    return lsym


def kernel(nodes, labels):
    lsym = _build_lsym(nodes)
    _, evecs = jnp.linalg.eigh(lsym)
    return evecs


# speedup vs baseline: 4.3886x; 1.2115x over previous
import functools

import jax
import jax.numpy as jnp
from jax.experimental import pallas as pl
from jax.experimental.pallas import tpu as pltpu

N = 2048
D = 256
K1 = 11
BLK = 256
NB = N // BLK
_BIG = 3.4e38


def _fused_kernel(nodes_ref, out_ref, w_s, degc_s, degr_s):
    p = pl.program_id(0)
    i = pl.program_id(1)
    row = pl.ds(i * BLK, BLK)

    @pl.when(p == 0)
    def _phase_w():
        nodes = nodes_ref[...]
        blk = nodes_ref[row, :]
        sq = jnp.sum(nodes * nodes, axis=1)
        sq_blk = jnp.sum(blk * blk, axis=1)
        g = jax.lax.dot_general(
            blk, nodes, (((1,), (1,)), ((), ())),
            preferred_element_type=jnp.float32,
            precision=jax.lax.Precision.HIGHEST,
        )
        s = sq_blk[:, None] + sq[None, :] - 2.0 * g
        s = jnp.maximum(s, 0.0)
        gi = jax.lax.broadcasted_iota(jnp.int32, (BLK, N), 0) + i * BLK
        gj = jax.lax.broadcasted_iota(jnp.int32, (BLK, N), 1)
        s = jnp.where(gi == gj, 0.0, s)
        cur = s
        for _ in range(K1 - 1):
            m = jnp.min(cur, axis=1, keepdims=True)
            cur = jnp.where(cur == m, _BIG, cur)
        thr = jnp.min(cur, axis=1, keepdims=True)
        w_s[row, :] = jnp.where(s <= thr, jnp.exp(s * -0.5), 0.0)

    @pl.when(p == 1)
    def _phase_sym():
        for j in range(NB):
            col = pl.ds(j * BLK, BLK)
            a_tile = jnp.maximum(w_s[row, col], w_s[col, row].T)
            w_s[row, col] = a_tile
        a_blk = w_s[row, :]
        degc_s[row, :] = jnp.sum(a_blk, axis=1, keepdims=True)

        @pl.when(i == 0)
        def _():
            degr_s[...] = jnp.sum(a_blk, axis=0, keepdims=True)

        @pl.when(i != 0)
        def _():
            degr_s[...] += jnp.sum(a_blk, axis=0, keepdims=True)

    @pl.when(p == 2)
    def _phase_lsym():
        a = w_s[row, :]
        degi = degc_s[row, :]
        dinv_i = 1.0 / jnp.sqrt(degi)
        dinv_j = 1.0 / jnp.sqrt(degr_s[...])
        gi = jax.lax.broadcasted_iota(jnp.int32, (BLK, N), 0) + i * BLK
        gj = jax.lax.broadcasted_iota(jnp.int32, (BLK, N), 1)
        l = jnp.where(gi == gj, degi - a, -a)
        m1 = (dinv_i * l) * dinv_j
        m2 = (dinv_j * l) * dinv_i
        out_ref[...] = 0.5 * (m1 + m2)


@functools.partial(jax.jit, static_argnames=("interpret",))
def _build_lsym(nodes, interpret=False):
    return pl.pallas_call(
        _fused_kernel,
        grid=(3, NB),
        in_specs=[pl.BlockSpec((N, D), lambda p, i: (0, 0))],
        out_specs=pl.BlockSpec((BLK, N), lambda p, i: (jnp.where(p == 2, i, 0), 0)),
        out_shape=jax.ShapeDtypeStruct((N, N), jnp.float32),
        scratch_shapes=[
            pltpu.VMEM((N, N), jnp.float32),
            pltpu.VMEM((N, 1), jnp.float32),
            pltpu.VMEM((1, N), jnp.float32),
        ],
        interpret=interpret,
    )(nodes)


def kernel(nodes, labels):
    lsym = _build_lsym(nodes)
    _, evecs = jnp.linalg.eigh(lsym)
    return evecs


# speedup vs baseline: 4.4801x; 1.0209x over previous
import functools

import jax
import jax.numpy as jnp
from jax.experimental import pallas as pl
from jax.experimental.pallas import tpu as pltpu

N = 2048
D = 256
K1 = 11
BLK = 256
NB = N // BLK
_BIG = 3.4e38


def _fused_kernel(nodes_ref, out_ref, s_s, thrc_s, thrr_s, degc_s, degr_s):
    p = pl.program_id(0)
    i = pl.program_id(1)
    row = pl.ds(i * BLK, BLK)

    @pl.when(p == 0)
    def _phase_s_thr():
        nodes = nodes_ref[...]
        blk = nodes_ref[row, :]
        sq = jnp.sum(nodes * nodes, axis=1)
        sq_blk = jnp.sum(blk * blk, axis=1)
        g = jax.lax.dot_general(
            blk, nodes, (((1,), (1,)), ((), ())),
            preferred_element_type=jnp.float32,
            precision=jax.lax.Precision.HIGHEST,
        )
        s = sq_blk[:, None] + sq[None, :] - 2.0 * g
        s = jnp.maximum(s, 0.0)
        gi = jax.lax.broadcasted_iota(jnp.int32, (BLK, N), 0) + i * BLK
        gj = jax.lax.broadcasted_iota(jnp.int32, (BLK, N), 1)
        s = jnp.where(gi == gj, 0.0, s)
        s_s[row, :] = s
        cur = s
        for _ in range(K1 - 1):
            m = jnp.min(cur, axis=1, keepdims=True)
            cur = jnp.where(cur == m, _BIG, cur)
        thr = jnp.min(cur, axis=1, keepdims=True)
        thrc_s[row, :] = thr
        ti = jax.lax.broadcasted_iota(jnp.int32, (BLK, BLK), 0)
        tj = jax.lax.broadcasted_iota(jnp.int32, (BLK, BLK), 1)
        diag = jnp.where(ti == tj, thr, _BIG)
        thrr_s[0:1, pl.ds(i * BLK, BLK)] = jnp.min(diag, axis=0, keepdims=True)

    @pl.when(p == 1)
    def _phase_a_deg():
        s = s_s[row, :]
        thr_i = thrc_s[row, :]
        thr_j = thrr_s[...]
        a = jnp.where(s <= jnp.maximum(thr_i, thr_j), jnp.exp(s * -0.5), 0.0)
        s_s[row, :] = a
        degc_s[row, :] = jnp.sum(a, axis=1, keepdims=True)

        @pl.when(i == 0)
        def _():
            degr_s[...] = jnp.sum(a, axis=0, keepdims=True)

        @pl.when(i != 0)
        def _():
            degr_s[...] += jnp.sum(a, axis=0, keepdims=True)

    @pl.when(p == 2)
    def _phase_lsym():
        a = s_s[row, :]
        degi = degc_s[row, :]
        dinv_i = 1.0 / jnp.sqrt(degi)
        dinv_j = 1.0 / jnp.sqrt(degr_s[...])
        gi = jax.lax.broadcasted_iota(jnp.int32, (BLK, N), 0) + i * BLK
        gj = jax.lax.broadcasted_iota(jnp.int32, (BLK, N), 1)
        l = jnp.where(gi == gj, degi - a, -a)
        m1 = (dinv_i * l) * dinv_j
        m2 = (dinv_j * l) * dinv_i
        out_ref[...] = 0.5 * (m1 + m2)


@functools.partial(jax.jit, static_argnames=("interpret",))
def _build_lsym(nodes, interpret=False):
    return pl.pallas_call(
        _fused_kernel,
        grid=(3, NB),
        in_specs=[pl.BlockSpec((N, D), lambda p, i: (0, 0))],
        out_specs=pl.BlockSpec((BLK, N), lambda p, i: (jnp.where(p == 2, i, 0), 0)),
        out_shape=jax.ShapeDtypeStruct((N, N), jnp.float32),
        scratch_shapes=[
            pltpu.VMEM((N, N), jnp.float32),
            pltpu.VMEM((N, 1), jnp.float32),
            pltpu.VMEM((1, N), jnp.float32),
            pltpu.VMEM((N, 1), jnp.float32),
            pltpu.VMEM((1, N), jnp.float32),
        ],
        interpret=interpret,
    )(nodes)


def kernel(nodes, labels):
    lsym = _build_lsym(nodes)
    _, evecs = jnp.linalg.eigh(lsym)
    return evecs


# speedup vs baseline: 4.6312x; 1.0337x over previous
import functools

import jax
import jax.numpy as jnp
from jax.experimental import pallas as pl
from jax.experimental.pallas import tpu as pltpu

N = 2048
D = 256
K1 = 11
BLK = 256
NB = N // BLK
_BIG = 3.4e38


def _fused_kernel(nodes_ref, out_ref, s_s, thrc_s, thrr_s, degc_s, degr_s):
    p = pl.program_id(0)
    i = pl.program_id(1)
    row = pl.ds(i * BLK, BLK)

    @pl.when(p == 0)
    def _phase_s_thr():
        nodes = nodes_ref[...]
        blk = nodes_ref[row, :]
        sq = jnp.sum(nodes * nodes, axis=1)
        sq_blk = jnp.sum(blk * blk, axis=1)
        g = jax.lax.dot_general(
            blk, nodes, (((1,), (1,)), ((), ())),
            preferred_element_type=jnp.float32,
            precision=jax.lax.Precision.DEFAULT,
        )
        s = sq_blk[:, None] + sq[None, :] - 2.0 * g
        s = jnp.maximum(s, 0.0)
        gi = jax.lax.broadcasted_iota(jnp.int32, (BLK, N), 0) + i * BLK
        gj = jax.lax.broadcasted_iota(jnp.int32, (BLK, N), 1)
        s = jnp.where(gi == gj, 0.0, s)
        s_s[row, :] = s
        cur = s
        for _ in range(K1 - 1):
            m = jnp.min(cur, axis=1, keepdims=True)
            cur = jnp.where(cur == m, _BIG, cur)
        thr = jnp.min(cur, axis=1, keepdims=True)
        thrc_s[row, :] = thr
        ti = jax.lax.broadcasted_iota(jnp.int32, (BLK, BLK), 0)
        tj = jax.lax.broadcasted_iota(jnp.int32, (BLK, BLK), 1)
        diag = jnp.where(ti == tj, thr, _BIG)
        thrr_s[0:1, pl.ds(i * BLK, BLK)] = jnp.min(diag, axis=0, keepdims=True)

    @pl.when(p == 1)
    def _phase_a_deg():
        s = s_s[row, :]
        thr_i = thrc_s[row, :]
        thr_j = thrr_s[...]
        a = jnp.where(s <= jnp.maximum(thr_i, thr_j), jnp.exp(s * -0.5), 0.0)
        s_s[row, :] = a
        degc_s[row, :] = jnp.sum(a, axis=1, keepdims=True)

        @pl.when(i == 0)
        def _():
            degr_s[...] = jnp.sum(a, axis=0, keepdims=True)

        @pl.when(i != 0)
        def _():
            degr_s[...] += jnp.sum(a, axis=0, keepdims=True)

    @pl.when(p == 2)
    def _phase_lsym():
        a = s_s[row, :]
        degi = degc_s[row, :]
        dinv_i = 1.0 / jnp.sqrt(degi)
        dinv_j = 1.0 / jnp.sqrt(degr_s[...])
        gi = jax.lax.broadcasted_iota(jnp.int32, (BLK, N), 0) + i * BLK
        gj = jax.lax.broadcasted_iota(jnp.int32, (BLK, N), 1)
        l = jnp.where(gi == gj, degi - a, -a)
        m1 = (dinv_i * l) * dinv_j
        m2 = (dinv_j * l) * dinv_i
        out_ref[...] = 0.5 * (m1 + m2)


@functools.partial(jax.jit, static_argnames=("interpret",))
def _build_lsym(nodes, interpret=False):
    return pl.pallas_call(
        _fused_kernel,
        grid=(3, NB),
        in_specs=[pl.BlockSpec((N, D), lambda p, i: (0, 0))],
        out_specs=pl.BlockSpec((BLK, N), lambda p, i: (jnp.where(p == 2, i, 0), 0)),
        out_shape=jax.ShapeDtypeStruct((N, N), jnp.float32),
        scratch_shapes=[
            pltpu.VMEM((N, N), jnp.float32),
            pltpu.VMEM((N, 1), jnp.float32),
            pltpu.VMEM((1, N), jnp.float32),
            pltpu.VMEM((N, 1), jnp.float32),
            pltpu.VMEM((1, N), jnp.float32),
        ],
        interpret=interpret,
    )(nodes)


def kernel(nodes, labels):
    lsym = _build_lsym(nodes)
    _, evecs = jnp.linalg.eigh(lsym)
    return evecs


# speedup vs baseline: 4.6828x; 1.0111x over previous
import functools

import jax
import jax.numpy as jnp
from jax.experimental import pallas as pl
from jax.experimental.pallas import tpu as pltpu

N = 2048
D = 256
K1 = 11
BLK = 512
NB = N // BLK
_BIG = 3.4e38


def _fused_kernel(nodes_ref, out_ref, s_s, thrc_s, thrr_s, degc_s, degr_s):
    p = pl.program_id(0)
    i = pl.program_id(1)
    row = pl.ds(i * BLK, BLK)

    @pl.when(p == 0)
    def _phase_s_thr():
        nodes = nodes_ref[...]
        blk = nodes_ref[row, :]
        sq = jnp.sum(nodes * nodes, axis=1)
        sq_blk = jnp.sum(blk * blk, axis=1)
        g = jax.lax.dot_general(
            blk, nodes, (((1,), (1,)), ((), ())),
            preferred_element_type=jnp.float32,
            precision=jax.lax.Precision.DEFAULT,
        )
        s = sq_blk[:, None] + sq[None, :] - 2.0 * g
        s = jnp.maximum(s, 0.0)
        gi = jax.lax.broadcasted_iota(jnp.int32, (BLK, N), 0) + i * BLK
        gj = jax.lax.broadcasted_iota(jnp.int32, (BLK, N), 1)
        s = jnp.where(gi == gj, 0.0, s)
        s_s[row, :] = s
        cur = s
        for _ in range(K1 - 1):
            m = jnp.min(cur, axis=1, keepdims=True)
            cur = jnp.where(cur == m, _BIG, cur)
        thr = jnp.min(cur, axis=1, keepdims=True)
        thrc_s[row, :] = thr
        ti = jax.lax.broadcasted_iota(jnp.int32, (BLK, BLK), 0)
        tj = jax.lax.broadcasted_iota(jnp.int32, (BLK, BLK), 1)
        diag = jnp.where(ti == tj, thr, _BIG)
        thrr_s[0:1, pl.ds(i * BLK, BLK)] = jnp.min(diag, axis=0, keepdims=True)

    @pl.when(p == 1)
    def _phase_a_deg():
        s = s_s[row, :]
        thr_i = thrc_s[row, :]
        thr_j = thrr_s[...]
        a = jnp.where(s <= jnp.maximum(thr_i, thr_j), jnp.exp(s * -0.5), 0.0)
        s_s[row, :] = a
        degc_s[row, :] = jnp.sum(a, axis=1, keepdims=True)

        @pl.when(i == 0)
        def _():
            degr_s[...] = jnp.sum(a, axis=0, keepdims=True)

        @pl.when(i != 0)
        def _():
            degr_s[...] += jnp.sum(a, axis=0, keepdims=True)

    @pl.when(p == 2)
    def _phase_lsym():
        a = s_s[row, :]
        degi = degc_s[row, :]
        dinv_i = 1.0 / jnp.sqrt(degi)
        dinv_j = 1.0 / jnp.sqrt(degr_s[...])
        gi = jax.lax.broadcasted_iota(jnp.int32, (BLK, N), 0) + i * BLK
        gj = jax.lax.broadcasted_iota(jnp.int32, (BLK, N), 1)
        l = jnp.where(gi == gj, degi - a, -a)
        m1 = (dinv_i * l) * dinv_j
        m2 = (dinv_j * l) * dinv_i
        out_ref[...] = 0.5 * (m1 + m2)


@functools.partial(jax.jit, static_argnames=("interpret",))
def _build_lsym(nodes, interpret=False):
    return pl.pallas_call(
        _fused_kernel,
        grid=(3, NB),
        in_specs=[pl.BlockSpec((N, D), lambda p, i: (0, 0))],
        out_specs=pl.BlockSpec((BLK, N), lambda p, i: (jnp.where(p == 2, i, 0), 0)),
        out_shape=jax.ShapeDtypeStruct((N, N), jnp.float32),
        scratch_shapes=[
            pltpu.VMEM((N, N), jnp.float32),
            pltpu.VMEM((N, 1), jnp.float32),
            pltpu.VMEM((1, N), jnp.float32),
            pltpu.VMEM((N, 1), jnp.float32),
            pltpu.VMEM((1, N), jnp.float32),
        ],
        interpret=interpret,
    )(nodes)


def kernel(nodes, labels):
    lsym = _build_lsym(nodes)
    _, evecs = jnp.linalg.eigh(lsym)
    return evecs
